# baseline (speedup 1.0000x reference)
import functools

import jax
import jax.numpy as jnp
from jax import lax
from jax.experimental import pallas as pl
from jax.experimental.pallas import tpu as pltpu
from jax.experimental.pallas import tpu_sc as plsc

P = 128
O = 16
NC, NS = 2, 16
NW = NC * NS
IDXW = 128


def _proj_body(e0_ref, e1_ref, w_ref, b_ref, t0_ref, t1_ref):
    dn = (((1,), (1,)), ((), ()))
    t0_ref[...] = lax.dot_general(
        e0_ref[...], w_ref[:, :P],
        dimension_numbers=dn,
        preferred_element_type=jnp.float32,
        precision=lax.Precision.HIGHEST,
    ) + b_ref[...]
    t1_ref[...] = lax.dot_general(
        e1_ref[...], w_ref[:, P:],
        dimension_numbers=dn,
        preferred_element_type=jnp.float32,
        precision=lax.Precision.HIGHEST,
    )


def _project_tables(E0, E1, W, b2d):
    V = E0.shape[0]
    return pl.pallas_call(
        _proj_body,
        out_shape=[
            jax.ShapeDtypeStruct((V, O), jnp.float32),
            jax.ShapeDtypeStruct((V, O), jnp.float32),
        ],
    )(E0, E1, W, b2d)


def _untile_body(r_ref, out_ref):
    chunks = [r_ref[:, pl.ds(16 * u, 16)] for u in range(8)]
    stacked = jnp.stack(chunks, axis=1)
    out_ref[...] = stacked.reshape(out_ref.shape)


def _untile(r, B):
    grid = 8
    rows = r.shape[0] // grid
    return pl.pallas_call(
        _untile_body,
        grid=(grid,),
        in_specs=[pl.BlockSpec((rows, 128), lambda i: (i, 0))],
        out_specs=pl.BlockSpec((rows * 8, O), lambda i: (i, 0)),
        out_shape=jax.ShapeDtypeStruct((B, O), jnp.float32),
    )(r)


@functools.lru_cache(maxsize=None)
def _make_sc_gather_add(B):
    bpw = B // NW
    nch = bpw // IDXW
    mesh = plsc.VectorSubcoreMesh(
        core_axis_name="c", subcore_axis_name="s",
        num_cores=NC, num_subcores=NS,
    )

    @functools.partial(
        pl.kernel,
        mesh=mesh,
        out_type=jax.ShapeDtypeStruct((B * O,), jnp.float32),
        scratch_types=[
            pltpu.VMEM((nch, IDXW), jnp.int32),
            pltpu.VMEM((nch, IDXW), jnp.int32),
            pltpu.VMEM((bpw, O), jnp.float32),
            pltpu.VMEM((bpw, O), jnp.float32),
            pltpu.VMEM((bpw * O,), jnp.float32),
            pltpu.SemaphoreType.DMA,
        ],
        compiler_params=pltpu.CompilerParams(use_tc_tiling_on_sc=False),
    )
    def sc_kernel(x0_hbm, x1_hbm, t0_hbm, t1_hbm, out_hbm,
                  i0_v, i1_v, r0_v, r1_v, of_v, sem):
        wid = lax.axis_index("s") * NC + lax.axis_index("c")
        rowbase = wid * nch
        pltpu.sync_copy(x0_hbm.at[pl.ds(rowbase, nch)], i0_v)
        pltpu.sync_copy(x1_hbm.at[pl.ds(rowbase, nch)], i1_v)
        copies = []
        for j in range(nch):
            dst = pl.ds(j * IDXW, IDXW)
            copies.append(pltpu.async_copy(
                t0_hbm.at[i0_v.at[j]], r0_v.at[dst], sem))
            copies.append(pltpu.async_copy(
                t1_hbm.at[i1_v.at[j]], r1_v.at[dst], sem))
        for c in copies:
            c.wait()

        @plsc.parallel_loop(0, bpw, 1, unroll=8)
        def add_row(i):
            of_v[pl.ds(i * O, O)] = r0_v[i] + r1_v[i]

        pltpu.sync_copy(of_v, out_hbm.at[pl.ds(wid * bpw * O, bpw * O)])

    return sc_kernel


def kernel(x, E0, E1, W, b):
    B = x.shape[0]
    assert B % (NW * IDXW) == 0
    t0, t1 = _project_tables(E0, E1, W, b.reshape(1, O))
    xi = x.astype(jnp.int32)
    x0 = xi[:, 0].reshape(B // IDXW, IDXW)
    x1 = xi[:, 1].reshape(B // IDXW, IDXW)
    r = _make_sc_gather_add(B)(x0, x1, t0, t1)
    return _untile(r.reshape(B * O // 128, 128), B)


# speedup vs baseline: 1.2771x; 1.2771x over previous
import functools

import jax
import jax.numpy as jnp
from jax import lax
from jax.experimental import pallas as pl
from jax.experimental.pallas import tpu as pltpu
from jax.experimental.pallas import tpu_sc as plsc

P = 128
O = 16
NC, NS = 2, 16
NW = NC * NS
IDXW = 128


def _proj_body(e0_ref, e1_ref, w_ref, b_ref, t0_ref, t1_ref):
    dn = (((1,), (1,)), ((), ()))
    t0_ref[...] = lax.dot_general(
        e0_ref[...], w_ref[:, :P],
        dimension_numbers=dn,
        preferred_element_type=jnp.float32,
        precision=lax.Precision.HIGHEST,
    ) + b_ref[...]
    t1_ref[...] = lax.dot_general(
        e1_ref[...], w_ref[:, P:],
        dimension_numbers=dn,
        preferred_element_type=jnp.float32,
        precision=lax.Precision.HIGHEST,
    )


def _project_tables(E0, E1, W, b2d):
    V = E0.shape[0]
    return pl.pallas_call(
        _proj_body,
        out_shape=[
            jax.ShapeDtypeStruct((V, O), jnp.float32),
            jax.ShapeDtypeStruct((V, O), jnp.float32),
        ],
    )(E0, E1, W, b2d)


def _untile_body(r_ref, out_ref):
    chunks = [r_ref[:, pl.ds(16 * u, 16)] for u in range(8)]
    stacked = jnp.stack(chunks, axis=1)
    out_ref[...] = stacked.reshape(out_ref.shape)


def _untile(r, B):
    grid = 8
    rows = r.shape[0] // grid
    return pl.pallas_call(
        _untile_body,
        grid=(grid,),
        in_specs=[pl.BlockSpec((rows, 128), lambda i: (i, 0))],
        out_specs=pl.BlockSpec((rows * 8, O), lambda i: (i, 0)),
        out_shape=jax.ShapeDtypeStruct((B, O), jnp.float32),
    )(r)


@functools.lru_cache(maxsize=None)
def _make_sc_gather_add(B):
    bpw = B // NW
    nch = bpw // IDXW
    mesh = plsc.VectorSubcoreMesh(
        core_axis_name="c", subcore_axis_name="s",
        num_cores=NC, num_subcores=NS,
    )

    @functools.partial(
        pl.kernel,
        mesh=mesh,
        out_type=jax.ShapeDtypeStruct((B * O,), jnp.float32),
        scratch_types=[
            pltpu.VMEM((nch, IDXW), jnp.int32),
            pltpu.VMEM((nch, IDXW), jnp.int32),
            pltpu.VMEM((bpw, O), jnp.float32),
            pltpu.VMEM((bpw, O), jnp.float32),
            pltpu.VMEM((bpw * O,), jnp.float32),
            pltpu.SemaphoreType.DMA,
        ],
        compiler_params=pltpu.CompilerParams(use_tc_tiling_on_sc=False),
    )
    def sc_kernel(x0_hbm, x1_hbm, t0_hbm, t1_hbm, out_hbm,
                  i0_v, i1_v, r0_v, r1_v, of_v, sem):
        wid = lax.axis_index("s") * NC + lax.axis_index("c")
        rowbase = wid * nch
        pltpu.sync_copy(x0_hbm.at[pl.ds(rowbase, nch)], i0_v)
        pltpu.sync_copy(x1_hbm.at[pl.ds(rowbase, nch)], i1_v)
        copies = []
        for j in range(nch):
            dst = pl.ds(j * IDXW, IDXW)
            copies.append(pltpu.async_copy(
                t0_hbm.at[i0_v.at[j]], r0_v.at[dst], sem))
            copies.append(pltpu.async_copy(
                t1_hbm.at[i1_v.at[j]], r1_v.at[dst], sem))
        for c in copies:
            c.wait()

        @plsc.parallel_loop(0, bpw, 1, unroll=8)
        def add_row(i):
            of_v[pl.ds(i * O, O)] = r0_v[i] + r1_v[i]

        pltpu.sync_copy(of_v, out_hbm.at[pl.ds(wid * bpw * O, bpw * O)])

    return sc_kernel


def kernel(x, E0, E1, W, b):
    B = x.shape[0]
    assert B % (NW * IDXW) == 0
    t0, t1 = _project_tables(E0, E1, W, b.reshape(1, O))
    xi = x.astype(jnp.int32)
    x0 = xi[:, 0].reshape(B // IDXW, IDXW)
    x1 = xi[:, 1].reshape(B // IDXW, IDXW)
    r = _make_sc_gather_add(B)(x0, x1, t0, t1)
    return r.reshape(B, O)
